# baseline (speedup 1.0000x reference)
import jax
import jax.numpy as jnp
from jax import lax
from jax.experimental import pallas as pl
from jax.experimental.pallas import tpu as pltpu


def _stage1_body(f_ref, mem_ref, gc_ref, fg_ref, idx_ref, rmax_ref,
                 cmax_ref, csum_ref,
                 idx_s, rmax_s, cmax_s, csum_s):
    pid = pl.program_id(0)
    nb = pl.num_programs(0)
    f = f_ref[...]
    D = f.shape[0]
    M = mem_ref.shape[0]
    colmean = jnp.mean(f, axis=2)
    a = colmean - jnp.max(colmean, axis=0, keepdims=True)
    e = jnp.exp(a)
    attn = e / jnp.sum(e, axis=0, keepdims=True)
    fg = lax.dot_general(attn, f, (((0,), (0,)), ((1,), (1,))),
                         preferred_element_type=jnp.float32) * (1.0 / D)
    fg_ref[...] = fg
    score = lax.dot_general(fg, mem_ref[...], (((1,), (1,)), ((), ())),
                            preferred_element_type=jnp.float32)
    rmax = jnp.max(score, axis=1, keepdims=True)
    es = jnp.exp(score - rmax)
    p = es / jnp.sum(es, axis=1, keepdims=True)
    ii = lax.broadcasted_iota(jnp.int32, score.shape, 1)
    idxv = jnp.min(jnp.where(score == rmax, ii, M), axis=1)
    idx_s[pl.ds(pid, 1), :] = idxv[None, :]
    rmax_s[pl.ds(pid, 1), :] = rmax[:, 0][None, :]
    resp = lax.dot_general(p, mem_ref[...], (((1,), (0,)), ((), ())),
                           preferred_element_type=jnp.float32)
    mr = fg + resp
    gc_ref[...] = f + mr[None, :, :]
    bm = jnp.max(score, axis=0, keepdims=True)

    @pl.when(pid == 0)
    def _():
        cmax_s[...] = bm
        csum_s[...] = jnp.sum(jnp.exp(score - bm), axis=0, keepdims=True)

    @pl.when(pid != 0)
    def _():
        m_old = cmax_s[...]
        m_new = jnp.maximum(m_old, bm)
        csum_s[...] = (csum_s[...] * jnp.exp(m_old - m_new)
                       + jnp.sum(jnp.exp(score - m_new), axis=0,
                                 keepdims=True))
        cmax_s[...] = m_new

    @pl.when(pid == nb - 1)
    def _():
        idx_ref[...] = idx_s[...]
        rmax_ref[...] = rmax_s[...]
        cmax_ref[...] = cmax_s[...]
        csum_ref[...] = csum_s[...]


def _stage1(ft, memory, bb):
    D, B, C = ft.shape
    M = memory.shape[0]
    nb = B // bb
    return pl.pallas_call(
        _stage1_body,
        grid=(nb,),
        in_specs=[
            pl.BlockSpec((D, bb, C), lambda i: (0, i, 0)),
            pl.BlockSpec((M, C), lambda i: (0, 0)),
        ],
        out_specs=[
            pl.BlockSpec((D, bb, C), lambda i: (0, i, 0)),
            pl.BlockSpec((bb, C), lambda i: (i, 0)),
            pl.BlockSpec((nb, bb), lambda i: (0, 0)),
            pl.BlockSpec((nb, bb), lambda i: (0, 0)),
            pl.BlockSpec((1, M), lambda i: (0, 0)),
            pl.BlockSpec((1, M), lambda i: (0, 0)),
        ],
        out_shape=[
            jax.ShapeDtypeStruct((D, B, C), jnp.float32),
            jax.ShapeDtypeStruct((B, C), jnp.float32),
            jax.ShapeDtypeStruct((nb, bb), jnp.int32),
            jax.ShapeDtypeStruct((nb, bb), jnp.float32),
            jax.ShapeDtypeStruct((1, M), jnp.float32),
            jax.ShapeDtypeStruct((1, M), jnp.float32),
        ],
        scratch_shapes=[
            pltpu.VMEM((nb, bb), jnp.int32),
            pltpu.VMEM((nb, bb), jnp.float32),
            pltpu.VMEM((1, M), jnp.float32),
            pltpu.VMEM((1, M), jnp.float32),
        ],
    )(ft, memory)


def _stage23_body(fg_ref, idx_ref, rmax_ref, cmax_ref, csum_ref, maskf_ref,
                  mem_ref, out_ref):
    B = fg_ref.shape[0]
    M = mem_ref.shape[0]
    idx = idx_ref[...].reshape(B, 1)
    oh = (idx == lax.broadcasted_iota(jnp.int32, (B, M), 1)).astype(
        jnp.float32)
    cmax_g = jnp.sum(oh * cmax_ref[...], axis=1)
    csum_g = jnp.sum(oh * csum_ref[...], axis=1)
    w = jnp.exp(rmax_ref[0, :] - cmax_g) / csum_g * maskf_ref[0, :]
    uv = fg_ref[...] * w[:, None]
    inc = lax.dot_general(oh, uv, (((0,), (0,)), ((), ())),
                          preferred_element_type=jnp.float32)
    um = inc + mem_ref[...]
    nrm = jnp.sqrt(jnp.sum(um * um, axis=1, keepdims=True))
    out_ref[...] = um / jnp.maximum(nrm, 1e-12)


def _stage23(fg, idx, rmax, cmax, csum, maskf, memory):
    M, C = memory.shape
    return pl.pallas_call(
        _stage23_body,
        out_shape=jax.ShapeDtypeStruct((M, C), jnp.float32),
    )(fg, idx, rmax, cmax, csum, maskf, memory)


def kernel(feature, memory, train, mask):
    B, C, D = feature.shape
    ft = jnp.transpose(feature, (2, 0, 1))
    maskf = (mask.astype(jnp.float32)
             * jnp.asarray(train, jnp.float32)).reshape(1, B)
    gct, fg, idx2, rmax2, cmax, csum = _stage1(ft, memory, 32)
    gc = jnp.transpose(gct, (1, 2, 0))
    idx = idx2.reshape(1, B)
    rmax = rmax2.reshape(1, B)
    upd = _stage23(fg, idx, rmax, cmax, csum, maskf, memory)
    return gc, upd
